# baseline (speedup 1.0000x reference)
import functools

import numpy as np
import jax
import jax.numpy as jnp
from jax import lax
from jax.experimental import pallas as pl
from jax.experimental.pallas import tpu as pltpu
from jax.experimental.pallas import tpu_sc as plsc

_LENGTH = 8192
_PROBABILITY = 0.12
_LANES = 16

_ROWS = 4096
_NC, _NS = 2, 16
_NW = _NC * _NS
_ROWS_PER_W = _ROWS // _NW
_G = 8
_Q = 8
_QW = _LENGTH // _Q
_CHUNK_SHAPE = (_G, _QW)
_NCH = (_ROWS_PER_W // _G) * _Q
_NBUF = 8
_LEAD = 4


def _threefry2x32_np(k0, k1, x0, x1):
    u32 = np.uint32
    rot_a, rot_b = (13, 15, 26, 6), (17, 29, 16, 24)
    ks = (u32(k0), u32(k1), u32(k0) ^ u32(k1) ^ u32(0x1BD11BDA))
    x0 = (x0 + ks[0]).astype(u32)
    x1 = (x1 + ks[1]).astype(u32)

    def rnd(x0, x1, r):
        x0 = (x0 + x1).astype(u32)
        x1 = ((x1 << u32(r)) | (x1 >> u32(32 - r))).astype(u32)
        return x0, x0 ^ x1

    schedule = ((rot_a, ks[1], ks[2], 1), (rot_b, ks[2], ks[0], 2),
                (rot_a, ks[0], ks[1], 3), (rot_b, ks[1], ks[2], 4),
                (rot_a, ks[2], ks[0], 5))
    for rots, a0, a1, i in schedule:
        for r in rots:
            x0, x1 = rnd(x0, x1, r)
        x0 = (x0 + a0).astype(u32)
        x1 = (x1 + a1 + u32(i)).astype(u32)
    return x0, x1


def _uniform_np(k0, k1, n):
    c1 = np.zeros(n, np.uint32)
    c2 = np.arange(n, dtype=np.uint32)
    o0, o1 = _threefry2x32_np(k0, k1, c1, c2)
    bits = o0 ^ o1
    fb = (bits >> np.uint32(9)) | np.uint32(0x3F800000)
    f = fb.view(np.float32) - np.float32(1.0)
    return np.maximum(np.float32(0.0), f)


def _jitter_pattern():
    s1, s2 = _threefry2x32_np(np.uint32(0), np.uint32(42),
                              np.zeros(2, np.uint32),
                              np.arange(2, dtype=np.uint32))
    k1 = (s1[0], s2[0])
    k2 = (s1[1], s2[1])
    replace = _uniform_np(k1[0], k1[1], _LENGTH) < np.float32(_PROBABILITY)
    direction = np.where(
        _uniform_np(k2[0], k2[1], _LENGTH) < np.float32(0.5), -1, 1)
    i = np.arange(_LENGTH)
    neighbor = np.where(
        i == 0, 1, np.where(i == _LENGTH - 1, _LENGTH - 2, i + direction))
    idx = np.where(replace, neighbor, i)
    return idx, replace


_IDX_H, _REPL_H = _jitter_pattern()
_DST0 = np.nonzero(_REPL_H)[0].astype(np.int32)
_SRC0 = _IDX_H[_DST0].astype(np.int32)


def _quarter_lists():
    srcs, dsts, offs, nvs = [], [], [], []
    off = 0
    for q in range(_Q):
        m = (_DST0 >= q * _QW) & (_DST0 < (q + 1) * _QW)
        s = (_SRC0[m] - q * _QW).astype(np.int32)
        d = (_DST0[m] - q * _QW).astype(np.int32)
        assert s.min() >= 0 and s.max() < _QW
        pad = (-len(d)) % _LANES
        if pad:
            s = np.concatenate([s, np.full(pad, s[-1], np.int32)])
            d = np.concatenate([d, np.full(pad, d[-1], np.int32)])
        srcs.append(s)
        dsts.append(d)
        offs.append(off)
        nvs.append(len(d) // _LANES)
        off += len(d)
    return (np.concatenate(srcs), np.concatenate(dsts), tuple(offs),
            tuple(nvs))


_SRC_ALL, _DST_ALL, _QOFF, _QNV = _quarter_lists()
_NIDX = len(_SRC_ALL)
_CBUF = max(_QNV) * _LANES


def _jitter_sc_body(x_hbm, src_hbm, dst_hbm, out_hbm, src_v, dst_v, cbuf,
                    *rest):
    bufs = rest[:_NBUF]
    in_sems = rest[_NBUF:2 * _NBUF]
    out_sems = rest[2 * _NBUF:3 * _NBUF]

    wid = lax.axis_index("s") * _NC + lax.axis_index("c")
    row0 = wid * _ROWS_PER_W

    pltpu.sync_copy(src_hbm, src_v)
    pltpu.sync_copy(dst_hbm, dst_v)

    def hbm_slice(ref, k):
        r = row0 + (k // _Q) * _G
        c = (k % _Q) * _QW
        return ref.at[pl.ds(r, _G), pl.ds(c, _QW)]

    def in_copy(k, b):
        return pltpu.make_async_copy(hbm_slice(x_hbm, k), bufs[b], in_sems[b])

    def out_copy(k, b):
        return pltpu.make_async_copy(bufs[b], hbm_slice(out_hbm, k),
                                     out_sems[b])

    def fix(b, q):
        buf = bufs[b]
        off = _QOFF[q]
        nv = _QNV[q]

        def row_pass(r, carry):
            rv = jnp.full((_LANES,), r, jnp.int32)

            def p1(c, carry2):
                sv = src_v[pl.ds(off + c * _LANES, _LANES)]
                cbuf[pl.ds(c * _LANES, _LANES)] = (
                    plsc.load_gather(buf, [rv, sv]))
                return carry2

            lax.fori_loop(0, nv, p1, 0, unroll=8)

            def p2(c, carry2):
                dv = dst_v[pl.ds(off + c * _LANES, _LANES)]
                plsc.store_scatter(buf, [rv, dv],
                                   cbuf[pl.ds(c * _LANES, _LANES)])
                return carry2

            lax.fori_loop(0, nv, p2, 0, unroll=8)
            return carry

        lax.fori_loop(0, _G, row_pass, 0)

    def step(k, p, wait_prev_out, start_next_in):
        bn = (p + _LEAD) % _NBUF
        if wait_prev_out:
            out_copy(k + _LEAD - _NBUF, bn).wait()
        if start_next_in:
            in_copy(k + _LEAD, bn).start()
        in_copy(k, p).wait()
        fix(p, p)
        out_copy(k, p).start()

    for k in range(_LEAD):
        in_copy(k, k % _NBUF).start()

    _H = _NBUF - _LEAD
    for k in range(_H):
        step(k, k % _NBUF, False, True)

    n_full = (_NCH - _LEAD) - _H
    n_mid = (n_full // _NBUF) * _NBUF

    def mid(g, carry):
        k0 = _H + g * _NBUF
        for r in range(_NBUF):
            step(k0 + r, (_H + r) % _NBUF, True, True)
        return carry

    lax.fori_loop(0, n_mid // _NBUF, mid, 0)

    for k in range(_H + n_mid, _NCH - _LEAD):
        step(k, k % _NBUF, True, True)
    for k in range(_NCH - _LEAD, _NCH):
        step(k, k % _NBUF, False, False)
    for k in range(_NCH - _NBUF, _NCH):
        out_copy(k, k % _NBUF).wait()


_SC_CALL = None


def _sc_call():
    global _SC_CALL
    if _SC_CALL is None:
        _SC_CALL = functools.partial(
            pl.kernel,
            out_type=jax.ShapeDtypeStruct((_ROWS, _LENGTH), jnp.float32),
            mesh=plsc.VectorSubcoreMesh(
                core_axis_name="c", subcore_axis_name="s",
                num_cores=_NC, num_subcores=_NS),
            scratch_types=[
                pltpu.VMEM((_NIDX,), jnp.int32),
                pltpu.VMEM((_NIDX,), jnp.int32),
                pltpu.VMEM((_CBUF,), jnp.float32),
            ] + [pltpu.VMEM(_CHUNK_SHAPE, jnp.float32)] * _NBUF
              + [pltpu.SemaphoreType.DMA] * (2 * _NBUF),
            compiler_params=pltpu.CompilerParams(needs_layout_passes=False),
        )(_jitter_sc_body)
    return _SC_CALL


def kernel(x):
    shape = x.shape
    out = _sc_call()(
        x.reshape(_ROWS, _LENGTH), jnp.asarray(_SRC_ALL),
        jnp.asarray(_DST_ALL))
    return out.reshape(shape)


# speedup vs baseline: 1.1113x; 1.1113x over previous
import functools

import numpy as np
import jax
import jax.numpy as jnp
from jax import lax
from jax.experimental import pallas as pl
from jax.experimental.pallas import tpu as pltpu
from jax.experimental.pallas import tpu_sc as plsc

_LENGTH = 8192
_PROBABILITY = 0.12
_LANES = 16

_ROWS = 4096
_NC, _NS = 2, 16
_NW = _NC * _NS
_ROWS_PER_W = _ROWS // _NW
_G = 8
_Q = 4
_QW = _LENGTH // _Q
_CHUNK_SHAPE = (_G, _QW)
_NCH = (_ROWS_PER_W // _G) * _Q
_NBUF = 4
_LEAD = 2


def _threefry2x32_np(k0, k1, x0, x1):
    u32 = np.uint32
    rot_a, rot_b = (13, 15, 26, 6), (17, 29, 16, 24)
    ks = (u32(k0), u32(k1), u32(k0) ^ u32(k1) ^ u32(0x1BD11BDA))
    x0 = (x0 + ks[0]).astype(u32)
    x1 = (x1 + ks[1]).astype(u32)

    def rnd(x0, x1, r):
        x0 = (x0 + x1).astype(u32)
        x1 = ((x1 << u32(r)) | (x1 >> u32(32 - r))).astype(u32)
        return x0, x0 ^ x1

    schedule = ((rot_a, ks[1], ks[2], 1), (rot_b, ks[2], ks[0], 2),
                (rot_a, ks[0], ks[1], 3), (rot_b, ks[1], ks[2], 4),
                (rot_a, ks[2], ks[0], 5))
    for rots, a0, a1, i in schedule:
        for r in rots:
            x0, x1 = rnd(x0, x1, r)
        x0 = (x0 + a0).astype(u32)
        x1 = (x1 + a1 + u32(i)).astype(u32)
    return x0, x1


def _uniform_np(k0, k1, n):
    c1 = np.zeros(n, np.uint32)
    c2 = np.arange(n, dtype=np.uint32)
    o0, o1 = _threefry2x32_np(k0, k1, c1, c2)
    bits = o0 ^ o1
    fb = (bits >> np.uint32(9)) | np.uint32(0x3F800000)
    f = fb.view(np.float32) - np.float32(1.0)
    return np.maximum(np.float32(0.0), f)


def _jitter_pattern():
    s1, s2 = _threefry2x32_np(np.uint32(0), np.uint32(42),
                              np.zeros(2, np.uint32),
                              np.arange(2, dtype=np.uint32))
    k1 = (s1[0], s2[0])
    k2 = (s1[1], s2[1])
    replace = _uniform_np(k1[0], k1[1], _LENGTH) < np.float32(_PROBABILITY)
    direction = np.where(
        _uniform_np(k2[0], k2[1], _LENGTH) < np.float32(0.5), -1, 1)
    i = np.arange(_LENGTH)
    neighbor = np.where(
        i == 0, 1, np.where(i == _LENGTH - 1, _LENGTH - 2, i + direction))
    idx = np.where(replace, neighbor, i)
    return idx, replace


_IDX_H, _REPL_H = _jitter_pattern()
_DST0 = np.nonzero(_REPL_H)[0].astype(np.int32)
_SRC0 = _IDX_H[_DST0].astype(np.int32)


def _quarter_lists():
    rows, srcs, dsts, offs, nvs = [], [], [], [], []
    off = 0
    for q in range(_Q):
        m = (_DST0 >= q * _QW) & (_DST0 < (q + 1) * _QW)
        s1 = (_SRC0[m] - q * _QW).astype(np.int32)
        d1 = (_DST0[m] - q * _QW).astype(np.int32)
        assert s1.min() >= 0 and s1.max() < _QW
        untouched = np.setdiff1d(np.arange(_QW, dtype=np.int32), d1)[0]
        r = np.repeat(np.arange(_G, dtype=np.int32), len(d1))
        s = np.tile(s1, _G)
        d = np.tile(d1, _G)
        pad = (-len(d)) % _LANES
        if pad:
            r = np.concatenate([r, np.full(pad, _G - 1, np.int32)])
            s = np.concatenate([s, np.full(pad, untouched, np.int32)])
            d = np.concatenate([d, np.full(pad, untouched, np.int32)])
        rows.append(r)
        srcs.append(s)
        dsts.append(d)
        offs.append(off)
        nvs.append(len(d) // _LANES)
        off += len(d)
    return (np.concatenate(rows), np.concatenate(srcs),
            np.concatenate(dsts), tuple(offs), tuple(nvs))


_ROW_ALL, _SRC_ALL, _DST_ALL, _QOFF, _QNV = _quarter_lists()
_NIDX = len(_SRC_ALL)


def _jitter_sc_body(x_hbm, row_hbm, src_hbm, dst_hbm, out_hbm,
                    row_v, src_v, dst_v, *rest):
    bufs = rest[:_NBUF]
    in_sems = rest[_NBUF:2 * _NBUF]
    out_sems = rest[2 * _NBUF:3 * _NBUF]

    wid = lax.axis_index("s") * _NC + lax.axis_index("c")
    row0 = wid * _ROWS_PER_W

    pltpu.sync_copy(row_hbm, row_v)
    pltpu.sync_copy(src_hbm, src_v)
    pltpu.sync_copy(dst_hbm, dst_v)

    def hbm_slice(ref, k):
        r = row0 + (k // _Q) * _G
        c = (k % _Q) * _QW
        return ref.at[pl.ds(r, _G), pl.ds(c, _QW)]

    def in_copy(k, b):
        return pltpu.make_async_copy(hbm_slice(x_hbm, k), bufs[b], in_sems[b])

    def out_copy(k, b):
        return pltpu.make_async_copy(bufs[b], hbm_slice(out_hbm, k),
                                     out_sems[b])

    def fix(b, q):
        buf = bufs[b]
        off = _QOFF[q]
        nv = _QNV[q]

        def ld(c):
            rv = row_v[pl.ds(off + c * _LANES, _LANES)]
            sv = src_v[pl.ds(off + c * _LANES, _LANES)]
            dv = dst_v[pl.ds(off + c * _LANES, _LANES)]
            return rv, dv, plsc.load_gather(buf, [rv, sv])

        r0, d0, v0 = ld(0)
        r1, d1, v1 = ld(1)

        def body(c, carry):
            ra, da, va, rb, db, vb = carry
            rv, dv, val = ld(c)
            plsc.store_scatter(buf, [ra, da], va)
            return rb, db, vb, rv, dv, val

        ra, da, va, rb, db, vb = lax.fori_loop(
            2, nv, body, (r0, d0, v0, r1, d1, v1), unroll=4)
        plsc.store_scatter(buf, [ra, da], va)
        plsc.store_scatter(buf, [rb, db], vb)

    def step(k, p, wait_prev_out, start_next_in):
        bn = (p + _LEAD) % _NBUF
        if wait_prev_out:
            out_copy(k + _LEAD - _NBUF, bn).wait()
        if start_next_in:
            in_copy(k + _LEAD, bn).start()
        in_copy(k, p).wait()
        fix(p, p)
        out_copy(k, p).start()

    for k in range(_LEAD):
        in_copy(k, k % _NBUF).start()

    _H = _NBUF - _LEAD
    for k in range(_H):
        step(k, k % _NBUF, False, True)

    n_full = (_NCH - _LEAD) - _H
    n_mid = (n_full // _NBUF) * _NBUF

    def mid(g, carry):
        k0 = _H + g * _NBUF
        for r in range(_NBUF):
            step(k0 + r, (_H + r) % _NBUF, True, True)
        return carry

    lax.fori_loop(0, n_mid // _NBUF, mid, 0)

    for k in range(_H + n_mid, _NCH - _LEAD):
        step(k, k % _NBUF, True, True)
    for k in range(_NCH - _LEAD, _NCH):
        step(k, k % _NBUF, False, False)
    for k in range(_NCH - _NBUF, _NCH):
        out_copy(k, k % _NBUF).wait()


_SC_CALL = None


def _sc_call():
    global _SC_CALL
    if _SC_CALL is None:
        _SC_CALL = functools.partial(
            pl.kernel,
            out_type=jax.ShapeDtypeStruct((_ROWS, _LENGTH), jnp.float32),
            mesh=plsc.VectorSubcoreMesh(
                core_axis_name="c", subcore_axis_name="s",
                num_cores=_NC, num_subcores=_NS),
            scratch_types=[
                pltpu.VMEM((_NIDX,), jnp.int32),
                pltpu.VMEM((_NIDX,), jnp.int32),
                pltpu.VMEM((_NIDX,), jnp.int32),
            ] + [pltpu.VMEM(_CHUNK_SHAPE, jnp.float32)] * _NBUF
              + [pltpu.SemaphoreType.DMA] * (2 * _NBUF),
            compiler_params=pltpu.CompilerParams(needs_layout_passes=False),
        )(_jitter_sc_body)
    return _SC_CALL


def kernel(x):
    shape = x.shape
    out = _sc_call()(
        x.reshape(_ROWS, _LENGTH), jnp.asarray(_ROW_ALL),
        jnp.asarray(_SRC_ALL), jnp.asarray(_DST_ALL))
    return out.reshape(shape)


# speedup vs baseline: 1.1250x; 1.0124x over previous
import functools

import numpy as np
import jax
import jax.numpy as jnp
from jax import lax
from jax.experimental import pallas as pl
from jax.experimental.pallas import tpu as pltpu
from jax.experimental.pallas import tpu_sc as plsc

_LENGTH = 8192
_PROBABILITY = 0.12
_LANES = 16

_ROWS = 4096
_NC, _NS = 2, 16
_NW = _NC * _NS
_ROWS_PER_W = _ROWS // _NW
_G = 8
_Q = 4
_QW = _LENGTH // _Q
_CHUNK_SHAPE = (_G, _QW)
_NCH = (_ROWS_PER_W // _G) * _Q
_NBUF = 4
_LEAD = 2


def _threefry2x32_np(k0, k1, x0, x1):
    u32 = np.uint32
    rot_a, rot_b = (13, 15, 26, 6), (17, 29, 16, 24)
    ks = (u32(k0), u32(k1), u32(k0) ^ u32(k1) ^ u32(0x1BD11BDA))
    x0 = (x0 + ks[0]).astype(u32)
    x1 = (x1 + ks[1]).astype(u32)

    def rnd(x0, x1, r):
        x0 = (x0 + x1).astype(u32)
        x1 = ((x1 << u32(r)) | (x1 >> u32(32 - r))).astype(u32)
        return x0, x0 ^ x1

    schedule = ((rot_a, ks[1], ks[2], 1), (rot_b, ks[2], ks[0], 2),
                (rot_a, ks[0], ks[1], 3), (rot_b, ks[1], ks[2], 4),
                (rot_a, ks[2], ks[0], 5))
    for rots, a0, a1, i in schedule:
        for r in rots:
            x0, x1 = rnd(x0, x1, r)
        x0 = (x0 + a0).astype(u32)
        x1 = (x1 + a1 + u32(i)).astype(u32)
    return x0, x1


def _uniform_np(k0, k1, n):
    c1 = np.zeros(n, np.uint32)
    c2 = np.arange(n, dtype=np.uint32)
    o0, o1 = _threefry2x32_np(k0, k1, c1, c2)
    bits = o0 ^ o1
    fb = (bits >> np.uint32(9)) | np.uint32(0x3F800000)
    f = fb.view(np.float32) - np.float32(1.0)
    return np.maximum(np.float32(0.0), f)


def _jitter_pattern():
    s1, s2 = _threefry2x32_np(np.uint32(0), np.uint32(42),
                              np.zeros(2, np.uint32),
                              np.arange(2, dtype=np.uint32))
    k1 = (s1[0], s2[0])
    k2 = (s1[1], s2[1])
    replace = _uniform_np(k1[0], k1[1], _LENGTH) < np.float32(_PROBABILITY)
    direction = np.where(
        _uniform_np(k2[0], k2[1], _LENGTH) < np.float32(0.5), -1, 1)
    i = np.arange(_LENGTH)
    neighbor = np.where(
        i == 0, 1, np.where(i == _LENGTH - 1, _LENGTH - 2, i + direction))
    idx = np.where(replace, neighbor, i)
    return idx, replace


_IDX_H, _REPL_H = _jitter_pattern()
_DST0 = np.nonzero(_REPL_H)[0].astype(np.int32)
_SRC0 = _IDX_H[_DST0].astype(np.int32)


def _quarter_lists():
    rows, srcs, dsts, offs, nvs = [], [], [], [], []
    off = 0
    for q in range(_Q):
        m = (_DST0 >= q * _QW) & (_DST0 < (q + 1) * _QW)
        s1 = (_SRC0[m] - q * _QW).astype(np.int32)
        d1 = (_DST0[m] - q * _QW).astype(np.int32)
        assert s1.min() >= 0 and s1.max() < _QW
        untouched = np.setdiff1d(np.arange(_QW, dtype=np.int32), d1)[0]
        r = np.repeat(np.arange(_G, dtype=np.int32), len(d1))
        s = np.tile(s1, _G)
        d = np.tile(d1, _G)
        pad = (-len(d)) % _LANES
        if pad:
            r = np.concatenate([r, np.full(pad, _G - 1, np.int32)])
            s = np.concatenate([s, np.full(pad, untouched, np.int32)])
            d = np.concatenate([d, np.full(pad, untouched, np.int32)])
        rows.append(r)
        srcs.append(s)
        dsts.append(d)
        offs.append(off)
        nvs.append(len(d) // _LANES)
        off += len(d)
    return (np.concatenate(rows), np.concatenate(srcs),
            np.concatenate(dsts), tuple(offs), tuple(nvs))


_ROW_ALL, _SRC_ALL, _DST_ALL, _QOFF, _QNV = _quarter_lists()
_NIDX = len(_SRC_ALL)


def _jitter_sc_body(x_hbm, row_hbm, src_hbm, dst_hbm, out_hbm,
                    row_v, src_v, dst_v, *rest):
    bufs = rest[:_NBUF]
    in_sems = rest[_NBUF:2 * _NBUF]
    out_sems = rest[2 * _NBUF:3 * _NBUF]

    wid = lax.axis_index("s") * _NC + lax.axis_index("c")
    row0 = wid * _ROWS_PER_W

    def hbm_slice(ref, k):
        r = row0 + (k // _Q) * _G
        c = (k % _Q) * _QW
        return ref.at[pl.ds(r, _G), pl.ds(c, _QW)]

    def in_copy(k, b):
        return pltpu.make_async_copy(hbm_slice(x_hbm, k), bufs[b], in_sems[b])

    def out_copy(k, b):
        return pltpu.make_async_copy(bufs[b], hbm_slice(out_hbm, k),
                                     out_sems[b])

    def fix(b, q):
        buf = bufs[b]
        off = _QOFF[q]
        nv = _QNV[q]

        def ld(c):
            rv = row_v[pl.ds(off + c * _LANES, _LANES)]
            sv = src_v[pl.ds(off + c * _LANES, _LANES)]
            dv = dst_v[pl.ds(off + c * _LANES, _LANES)]
            return rv, dv, plsc.load_gather(buf, [rv, sv])

        r0, d0, v0 = ld(0)
        r1, d1, v1 = ld(1)

        def body(c, carry):
            ra, da, va, rb, db, vb = carry
            rv, dv, val = ld(c)
            plsc.store_scatter(buf, [ra, da], va)
            return rb, db, vb, rv, dv, val

        ra, da, va, rb, db, vb = lax.fori_loop(
            2, nv, body, (r0, d0, v0, r1, d1, v1), unroll=8)
        plsc.store_scatter(buf, [ra, da], va)
        plsc.store_scatter(buf, [rb, db], vb)

    def step(k, p, wait_prev_out, start_next_in):
        bn = (p + _LEAD) % _NBUF
        if wait_prev_out:
            out_copy(k + _LEAD - _NBUF, bn).wait()
        if start_next_in:
            in_copy(k + _LEAD, bn).start()
        in_copy(k, p).wait()
        fix(p, p)
        out_copy(k, p).start()

    for k in range(_LEAD):
        in_copy(k, k % _NBUF).start()
    pltpu.sync_copy(row_hbm, row_v)
    pltpu.sync_copy(src_hbm, src_v)
    pltpu.sync_copy(dst_hbm, dst_v)

    _H = _NBUF - _LEAD
    for k in range(_H):
        step(k, k % _NBUF, False, True)

    n_full = (_NCH - _LEAD) - _H
    n_mid = (n_full // _NBUF) * _NBUF

    def mid(g, carry):
        k0 = _H + g * _NBUF
        for r in range(_NBUF):
            step(k0 + r, (_H + r) % _NBUF, True, True)
        return carry

    lax.fori_loop(0, n_mid // _NBUF, mid, 0)

    for k in range(_H + n_mid, _NCH - _LEAD):
        step(k, k % _NBUF, True, True)
    for k in range(_NCH - _LEAD, _NCH):
        step(k, k % _NBUF, False, False)
    for k in range(_NCH - _NBUF, _NCH):
        out_copy(k, k % _NBUF).wait()


_SC_CALL = None


def _sc_call():
    global _SC_CALL
    if _SC_CALL is None:
        _SC_CALL = functools.partial(
            pl.kernel,
            out_type=jax.ShapeDtypeStruct((_ROWS, _LENGTH), jnp.float32),
            mesh=plsc.VectorSubcoreMesh(
                core_axis_name="c", subcore_axis_name="s",
                num_cores=_NC, num_subcores=_NS),
            scratch_types=[
                pltpu.VMEM((_NIDX,), jnp.int32),
                pltpu.VMEM((_NIDX,), jnp.int32),
                pltpu.VMEM((_NIDX,), jnp.int32),
            ] + [pltpu.VMEM(_CHUNK_SHAPE, jnp.float32)] * _NBUF
              + [pltpu.SemaphoreType.DMA] * (2 * _NBUF),
            compiler_params=pltpu.CompilerParams(needs_layout_passes=False),
        )(_jitter_sc_body)
    return _SC_CALL


def kernel(x):
    shape = x.shape
    out = _sc_call()(
        x.reshape(_ROWS, _LENGTH), jnp.asarray(_ROW_ALL),
        jnp.asarray(_SRC_ALL), jnp.asarray(_DST_ALL))
    return out.reshape(shape)
